# baseline (speedup 1.0000x reference)
import functools

import jax
import jax.numpy as jnp
from jax import lax
from jax.experimental import pallas as pl
from jax.experimental.pallas import tpu as pltpu
from jax.experimental.pallas import tpu_sc as plsc

NUM_CORES = 2
NUM_SUBCORES = 16
NUM_WORKERS = NUM_CORES * NUM_SUBCORES
LANES = 16
NBUF = 4
BATCH_GROUPS = 4
ROW_GROUPS = NUM_WORKERS // BATCH_GROUPS
CHUNKS = 3


def _sc_add(B, N, D, p_hbm, t_hbm, o_hbm, *refs):
    wrows = N // ROW_GROUPS
    crows = wrows // CHUNKS
    bpw = B // BATCH_GROUPS

    tbl_v = refs[0]
    bufs = refs[1 : 1 + NBUF]
    isems = refs[1 + NBUF : 1 + 2 * NBUF]
    osems = refs[1 + 2 * NBUF : 1 + 3 * NBUF]

    wid = lax.axis_index("s") * NUM_CORES + lax.axis_index("c")
    bg = wid // ROW_GROUPS
    rg = wid % ROW_GROUPS
    r0 = rg * wrows
    b0 = bg * bpw
    pltpu.sync_copy(t_hbm.at[pl.ds(r0, wrows)], tbl_v)

    def in_slice(b, c):
        return p_hbm.at[b0 + b, pl.ds(r0 + c * crows, crows)]

    def out_slice(b, c):
        return o_hbm.at[b0 + b, pl.ds(r0 + c * crows, crows)]

    def add(k, c):
        buf = bufs[k]

        @plsc.parallel_loop(0, crows)
        def _(r):
            tr = c * crows + r
            for v in range(D // LANES):
                s = pl.ds(v * LANES, LANES)
                plsc.addupdate(buf.at[r, s], tbl_v[tr, s])

    for c in range(CHUNKS):
        for k in range(NBUF // 2):
            pltpu.async_copy(in_slice(k, c), bufs[k], isems[k])

        def group(g, carry):
            for k in range(NBUF):
                b = g * NBUF + k
                pltpu.make_async_copy(in_slice(b, c), bufs[k], isems[k]).wait()
                add(k, c)
                ko = (k + NBUF // 2) % NBUF

                def drain_out():
                    pltpu.make_async_copy(
                        bufs[ko], out_slice(b - NBUF // 2, c), osems[ko]
                    ).wait()

                if k >= NBUF // 2:
                    drain_out()
                else:
                    pl.when(g > 0)(drain_out)
                pltpu.async_copy(bufs[k], out_slice(b, c), osems[k])

                def prefetch_in():
                    pltpu.async_copy(in_slice(b + NBUF // 2, c), bufs[ko], isems[ko])

                if k < NBUF // 2:
                    prefetch_in()
                else:
                    pl.when(g < bpw // NBUF - 1)(prefetch_in)
            return carry

        lax.fori_loop(0, bpw // NBUF, group, 0)

        for k in range(NBUF // 2, NBUF):
            b = bpw - NBUF + k
            pltpu.make_async_copy(bufs[k], out_slice(b, c), osems[k]).wait()


def _tc_add_body(p_ref, t_ref, o_ref):
    o_ref[...] = p_ref[...] + t_ref[...]


def _tc_add(patches, table, b_start, BB=4):
    B, N, D = patches.shape
    nb = B - b_start
    return pl.pallas_call(
        _tc_add_body,
        grid=(nb // BB,),
        in_specs=[
            pl.BlockSpec((BB, N, D), lambda i: (i + b_start // BB, 0, 0)),
            pl.BlockSpec((N, D), lambda i: (0, 0)),
        ],
        out_specs=pl.BlockSpec((BB, N, D), lambda i: (i, 0, 0)),
        out_shape=jax.ShapeDtypeStruct((nb, N, D), patches.dtype),
    )(patches, table)


B_SC = 48


def kernel(patches, pos_table):
    B, N, D = patches.shape
    table = pos_table[:N]
    wrows = N // ROW_GROUPS
    crows = wrows // CHUNKS

    mesh = plsc.VectorSubcoreMesh(core_axis_name="c", subcore_axis_name="s")
    scratch = (
        [pltpu.VMEM((wrows, D), jnp.float32)]
        + [pltpu.VMEM((crows, D), jnp.float32) for _ in range(NBUF)]
        + [pltpu.SemaphoreType.DMA for _ in range(2 * NBUF)]
    )
    sc_call = functools.partial(
        pl.kernel,
        out_type=jax.ShapeDtypeStruct((B_SC, N, D), jnp.float32),
        mesh=mesh,
        scratch_types=scratch,
    )(functools.partial(_sc_add, B_SC, N, D))
    sc_out = sc_call(patches, table)
    tc_out = _tc_add(patches, table, B_SC)
    return jnp.concatenate([sc_out, tc_out], axis=0)


# speedup vs baseline: 1.5379x; 1.5379x over previous
import functools

import jax
import jax.numpy as jnp
from jax import lax
from jax.experimental import pallas as pl
from jax.experimental.pallas import tpu as pltpu
from jax.experimental.pallas import tpu_sc as plsc

NUM_CORES = 2
NUM_SUBCORES = 16
NUM_WORKERS = NUM_CORES * NUM_SUBCORES
LANES = 16
NBUF = 4
BATCH_GROUPS = 4
ROW_GROUPS = NUM_WORKERS // BATCH_GROUPS
CHUNKS = 3


def _sc_add(B, N, D, p_hbm, t_hbm, o_hbm, *refs):
    wrows = N // ROW_GROUPS
    crows = wrows // CHUNKS
    bpw = B // BATCH_GROUPS

    tbl_v = refs[0]
    bufs = refs[1 : 1 + NBUF]
    isems = refs[1 + NBUF : 1 + 2 * NBUF]
    osems = refs[1 + 2 * NBUF : 1 + 3 * NBUF]

    wid = lax.axis_index("s") * NUM_CORES + lax.axis_index("c")
    bg = wid // ROW_GROUPS
    rg = wid % ROW_GROUPS
    r0 = rg * wrows
    b0 = bg * bpw
    pltpu.sync_copy(t_hbm.at[pl.ds(r0, wrows)], tbl_v)

    def in_slice(b, c):
        return p_hbm.at[b0 + b, pl.ds(r0 + c * crows, crows)]

    def out_slice(b, c):
        return o_hbm.at[b0 + b, pl.ds(r0 + c * crows, crows)]

    def add(k, c):
        buf = bufs[k]

        @plsc.parallel_loop(0, crows)
        def _(r):
            tr = c * crows + r
            for v in range(D // LANES):
                s = pl.ds(v * LANES, LANES)
                plsc.addupdate(buf.at[r, s], tbl_v[tr, s])

    for c in range(CHUNKS):
        for k in range(NBUF // 2):
            pltpu.async_copy(in_slice(k, c), bufs[k], isems[k])

        def group(g, carry):
            for k in range(NBUF):
                b = g * NBUF + k
                pltpu.make_async_copy(in_slice(b, c), bufs[k], isems[k]).wait()
                ko = (k + NBUF // 2) % NBUF

                def drain_out():
                    pltpu.make_async_copy(
                        bufs[ko], out_slice(b - NBUF // 2, c), osems[ko]
                    ).wait()

                if k >= NBUF // 2:
                    drain_out()
                else:
                    pl.when(g > 0)(drain_out)

                def prefetch_in():
                    pltpu.async_copy(in_slice(b + NBUF // 2, c), bufs[ko], isems[ko])

                if k < NBUF // 2:
                    prefetch_in()
                else:
                    pl.when(g < bpw // NBUF - 1)(prefetch_in)
                add(k, c)
                pltpu.async_copy(bufs[k], out_slice(b, c), osems[k])
            return carry

        lax.fori_loop(0, bpw // NBUF, group, 0)

        for k in range(NBUF // 2, NBUF):
            b = bpw - NBUF + k
            pltpu.make_async_copy(bufs[k], out_slice(b, c), osems[k]).wait()


def _tc_add_body(p_ref, t_ref, o_ref):
    o_ref[...] = p_ref[...] + t_ref[...]


def _tc_add(patches, table, b_start, BB=4):
    B, N, D = patches.shape
    nb = B - b_start
    return pl.pallas_call(
        _tc_add_body,
        grid=(nb // BB,),
        in_specs=[
            pl.BlockSpec((BB, N, D), lambda i: (i + b_start // BB, 0, 0)),
            pl.BlockSpec((N, D), lambda i: (0, 0)),
        ],
        out_specs=pl.BlockSpec((BB, N, D), lambda i: (i, 0, 0)),
        out_shape=jax.ShapeDtypeStruct((nb, N, D), patches.dtype),
    )(patches, table)


B_SC = 48


def kernel(patches, pos_table):
    B, N, D = patches.shape
    table = pos_table[:N]
    wrows = N // ROW_GROUPS
    crows = wrows // CHUNKS

    mesh = plsc.VectorSubcoreMesh(core_axis_name="c", subcore_axis_name="s")
    scratch = (
        [pltpu.VMEM((wrows, D), jnp.float32)]
        + [pltpu.VMEM((crows, D), jnp.float32) for _ in range(NBUF)]
        + [pltpu.SemaphoreType.DMA for _ in range(2 * NBUF)]
    )
    sc_call = functools.partial(
        pl.kernel,
        out_type=jax.ShapeDtypeStruct((B, N, D), jnp.float32),
        mesh=mesh,
        scratch_types=scratch,
    )(functools.partial(_sc_add, B, N, D))
    return sc_call(patches, table)


# speedup vs baseline: 1.8812x; 1.2232x over previous
import functools

import jax
import jax.numpy as jnp
from jax import lax
from jax.experimental import pallas as pl
from jax.experimental.pallas import tpu as pltpu
from jax.experimental.pallas import tpu_sc as plsc

NUM_CORES = 2
NUM_SUBCORES = 16
NUM_WORKERS = NUM_CORES * NUM_SUBCORES
LOOKUP_ROWS = 24


def _sc_lookup(N, D, pos_start, t_hbm, emb_hbm, row_v):
    wid = lax.axis_index("s") * NUM_CORES + lax.axis_index("c")
    nw = N // LOOKUP_ROWS

    @pl.when(wid < nw)
    def _():
        r0 = wid * LOOKUP_ROWS
        pltpu.sync_copy(t_hbm.at[pl.ds(pos_start + r0, LOOKUP_ROWS)], row_v)
        pltpu.sync_copy(row_v, emb_hbm.at[pl.ds(r0, LOOKUP_ROWS)])


def _tc_add_body(p_ref, t_ref, o_ref):
    o_ref[...] = p_ref[...] + t_ref[...]


def kernel(patches, pos_table):
    B, N, D = patches.shape
    pos_start = pos_table.shape[0] - N

    mesh = plsc.VectorSubcoreMesh(core_axis_name="c", subcore_axis_name="s")
    lookup = functools.partial(
        pl.kernel,
        out_type=jax.ShapeDtypeStruct((N, D), pos_table.dtype),
        mesh=mesh,
        scratch_types=[pltpu.VMEM((LOOKUP_ROWS, D), pos_table.dtype)],
    )(functools.partial(_sc_lookup, N, D, pos_start))
    pos_emb = lookup(pos_table)

    BB = 4
    return pl.pallas_call(
        _tc_add_body,
        grid=(B // BB,),
        in_specs=[
            pl.BlockSpec((BB, N, D), lambda i: (i, 0, 0)),
            pl.BlockSpec((N, D), lambda i: (0, 0)),
        ],
        out_specs=pl.BlockSpec((BB, N, D), lambda i: (i, 0, 0)),
        out_shape=jax.ShapeDtypeStruct((B, N, D), patches.dtype),
    )(patches, pos_emb)


# speedup vs baseline: 1.8988x; 1.0094x over previous
import functools

import jax
import jax.numpy as jnp
from jax import lax
from jax.experimental import pallas as pl
from jax.experimental.pallas import tpu as pltpu
from jax.experimental.pallas import tpu_sc as plsc

NUM_CORES = 2
NUM_SUBCORES = 16
NUM_WORKERS = NUM_CORES * NUM_SUBCORES
LOOKUP_ROWS = 24


def _sc_lookup(N, D, pos_start, t_hbm, emb_hbm, row_v):
    wid = lax.axis_index("s") * NUM_CORES + lax.axis_index("c")
    nw = N // LOOKUP_ROWS

    @pl.when(wid < nw)
    def _():
        r0 = wid * LOOKUP_ROWS
        pltpu.sync_copy(t_hbm.at[pl.ds(pos_start + r0, LOOKUP_ROWS)], row_v)
        pltpu.sync_copy(row_v, emb_hbm.at[pl.ds(r0, LOOKUP_ROWS)])


def _tc_add_body(p_ref, t_ref, o_ref):
    o_ref[...] = p_ref[...] + t_ref[...]


def kernel(patches, pos_table):
    B, N, D = patches.shape
    pos_start = pos_table.shape[0] - N

    mesh = plsc.VectorSubcoreMesh(core_axis_name="c", subcore_axis_name="s")
    lookup = functools.partial(
        pl.kernel,
        out_type=jax.ShapeDtypeStruct((N, D), pos_table.dtype),
        mesh=mesh,
        scratch_types=[pltpu.VMEM((LOOKUP_ROWS, D), pos_table.dtype)],
    )(functools.partial(_sc_lookup, N, D, pos_start))
    pos_emb = lookup(pos_table)

    BB = 8
    return pl.pallas_call(
        _tc_add_body,
        grid=(B // BB,),
        in_specs=[
            pl.BlockSpec((BB, N, D), lambda i: (i, 0, 0)),
            pl.BlockSpec((N, D), lambda i: (0, 0)),
        ],
        out_specs=pl.BlockSpec((BB, N, D), lambda i: (i, 0, 0)),
        out_shape=jax.ShapeDtypeStruct((B, N, D), patches.dtype),
    )(patches, pos_emb)


# speedup vs baseline: 1.9054x; 1.0035x over previous
import functools

import jax
import jax.numpy as jnp
from jax import lax
from jax.experimental import pallas as pl
from jax.experimental.pallas import tpu as pltpu
from jax.experimental.pallas import tpu_sc as plsc

NUM_CORES = 2
NUM_SUBCORES = 16
NUM_WORKERS = NUM_CORES * NUM_SUBCORES
LOOKUP_ROWS = 24


def _sc_lookup(N, D, pos_start, t_hbm, emb_hbm, row_v):
    cid = lax.axis_index("c")
    half = N // NUM_CORES
    r0 = cid * half
    pltpu.sync_copy(t_hbm.at[pl.ds(pos_start + r0, half)], row_v)
    pltpu.sync_copy(row_v, emb_hbm.at[pl.ds(r0, half)])


def _tc_add_body(p_ref, t_ref, o_ref):
    o_ref[...] = p_ref[...] + t_ref[...]


def kernel(patches, pos_table):
    B, N, D = patches.shape
    pos_start = pos_table.shape[0] - N

    mesh = plsc.ScalarSubcoreMesh(axis_name="c", num_cores=NUM_CORES)
    lookup = functools.partial(
        pl.kernel,
        out_type=jax.ShapeDtypeStruct((N, D), pos_table.dtype),
        mesh=mesh,
        scratch_types=[pltpu.VMEM_SHARED((N // NUM_CORES, D), pos_table.dtype)],
    )(functools.partial(_sc_lookup, N, D, pos_start))
    pos_emb = lookup(pos_table)

    BB = 8
    return pl.pallas_call(
        _tc_add_body,
        grid=(B // BB,),
        in_specs=[
            pl.BlockSpec((BB, N, D), lambda i: (i, 0, 0)),
            pl.BlockSpec((N, D), lambda i: (0, 0)),
        ],
        out_specs=pl.BlockSpec((BB, N, D), lambda i: (i, 0, 0)),
        out_shape=jax.ShapeDtypeStruct((B, N, D), patches.dtype),
    )(patches, pos_emb)
